# baseline (speedup 1.0000x reference)
import jax
import jax.numpy as jnp
from jax.experimental import pallas as pl
from jax.experimental.pallas import tpu as pltpu

NSEG = 10000
SWIN = 64
ACC_ROWS = 10112


def _pick_block_rows(n):
    for r in (1280, 640, 320, 160, 80, 40, 8):
        if n % r == 0:
            return r
    return n


def _body(x_ref, ids_v_ref, ids_s_ref, w_ref, b_ref, out_ref, acc_ref):
    i = pl.program_id(0)
    nblk = pl.num_programs(0)
    R = x_ref.shape[0]
    D = x_ref.shape[1]

    @pl.when(i == 0)
    def _init():
        acc_ref[...] = jnp.zeros_like(acc_ref)

    feat = jax.lax.dot_general(
        x_ref[...].astype(jnp.bfloat16), w_ref[...].astype(jnp.bfloat16),
        (((1,), (1,)), ((), ())),
        preferred_element_type=jnp.float32)
    feat = feat + b_ref[...]
    featcat = jnp.concatenate(
        [feat, feat * feat, jnp.ones((R, D), jnp.float32)],
        axis=1).astype(jnp.bfloat16)

    ids_v = ids_v_ref[0]
    s_lo = ids_s_ref[0, 0, 0]
    s_hi = ids_s_ref[0, 0, R - 1]
    a0 = (s_lo // 8) * 8

    def win_body(base):
        base = pl.multiple_of(base, 8)
        rel = ids_v - base
        sidx = jax.lax.broadcasted_iota(jnp.int32, (SWIN, 1), 0)
        P = jnp.where(rel == sidx, 1.0, 0.0)
        part = jax.lax.dot_general(
            P.astype(jnp.bfloat16), featcat, (((1,), (0,)), ((), ())),
            preferred_element_type=jnp.float32)
        acc_ref[pl.ds(base, SWIN), :] += part
        return base + SWIN

    jax.lax.while_loop(lambda b: b <= s_hi, win_body, a0)

    @pl.when(i == nblk - 1)
    def _fin():
        sums = acc_ref[:NSEG, 0:D]
        sumsq = acc_ref[:NSEG, D:2 * D]
        cnt = acc_ref[:NSEG, 2 * D:2 * D + 1]
        cnt_c = jnp.maximum(cnt, 1.0)
        mean = sums / cnt_c
        sq = jnp.maximum(sumsq - sums * mean, 0.0)
        cnt_u = jnp.maximum(cnt - 1.0, 1.0)
        std = jnp.sqrt(sq / (cnt_u + 1e-6))
        out_ref[:, 0:D] = mean
        out_ref[:, D:2 * D] = std


def kernel(x, batch, W, b):
    N, D = x.shape
    R = _pick_block_rows(N)
    nblk = N // R
    ids = batch.astype(jnp.int32)
    ids3 = ids.reshape(nblk, 1, R)
    ids2 = ids.reshape(nblk, 1, R)
    b2 = b.reshape(1, D)
    return pl.pallas_call(
        _body,
        grid=(nblk,),
        in_specs=[
            pl.BlockSpec((R, D), lambda i: (i, 0)),
            pl.BlockSpec((1, 1, R), lambda i: (i, 0, 0)),
            pl.BlockSpec((1, 1, R), lambda i: (i, 0, 0),
                         memory_space=pltpu.SMEM),
            pl.BlockSpec((D, D), lambda i: (0, 0)),
            pl.BlockSpec((1, D), lambda i: (0, 0)),
        ],
        out_specs=pl.BlockSpec((NSEG, 2 * D), lambda i: (0, 0)),
        out_shape=jax.ShapeDtypeStruct((NSEG, 2 * D), jnp.float32),
        scratch_shapes=[pltpu.VMEM((ACC_ROWS, 3 * D), jnp.float32)],
        compiler_params=pltpu.CompilerParams(
            dimension_semantics=("arbitrary",)),
    )(x, ids3, ids2, W, b2)


# speedup vs baseline: 1.4561x; 1.4561x over previous
import jax
import jax.numpy as jnp
from jax.experimental import pallas as pl
from jax.experimental.pallas import tpu as pltpu

NSEG = 10000
SWIN = 96
ACC_ROWS = 10112


def _pick_block_rows(n):
    for r in (2560, 1280, 640, 320, 160, 80, 40, 8):
        if n % r == 0:
            return r
    return n


def _body(x_ref, ids_v_ref, ids_s_ref, w_ref, b_ref, out_ref, acc_ref):
    i = pl.program_id(0)
    nblk = pl.num_programs(0)
    R = x_ref.shape[0]
    D = x_ref.shape[1]

    @pl.when(i == 0)
    def _init():
        acc_ref[...] = jnp.zeros_like(acc_ref)

    feat = jax.lax.dot_general(
        x_ref[...].astype(jnp.bfloat16), w_ref[...].astype(jnp.bfloat16),
        (((1,), (1,)), ((), ())),
        preferred_element_type=jnp.float32)
    feat = feat + b_ref[...]
    featcat = jnp.concatenate(
        [feat, feat * feat, jnp.ones((R, D), jnp.float32)],
        axis=1).astype(jnp.bfloat16)

    ids_v = ids_v_ref[0]
    s_lo = ids_s_ref[0, 0, 0]
    s_hi = ids_s_ref[0, 0, 1]
    a0 = (s_lo // 8) * 8

    def win_body(base):
        base = pl.multiple_of(base, 8)
        rel = ids_v - base
        sidx = jax.lax.broadcasted_iota(jnp.int32, (SWIN, 1), 0)
        P = jnp.where(rel == sidx, 1.0, 0.0)
        part = jax.lax.dot_general(
            P.astype(jnp.bfloat16), featcat, (((1,), (0,)), ((), ())),
            preferred_element_type=jnp.float32)
        acc_ref[pl.ds(base, SWIN), :] += part
        return base + SWIN

    jax.lax.while_loop(lambda b: b <= s_hi, win_body, a0)

    @pl.when(i == nblk - 1)
    def _fin():
        sums = acc_ref[:NSEG, 0:D]
        sumsq = acc_ref[:NSEG, D:2 * D]
        cnt = acc_ref[:NSEG, 2 * D:2 * D + 1]
        cnt_c = jnp.maximum(cnt, 1.0)
        mean = sums / cnt_c
        sq = jnp.maximum(sumsq - sums * mean, 0.0)
        cnt_u = jnp.maximum(cnt - 1.0, 1.0)
        std = jnp.sqrt(sq / (cnt_u + 1e-6))
        out_ref[:, 0:D] = mean
        out_ref[:, D:2 * D] = std


def kernel(x, batch, W, b):
    N, D = x.shape
    R = _pick_block_rows(N)
    nblk = N // R
    ids = batch.astype(jnp.int32)
    ids3 = ids.reshape(nblk, 1, R)
    ids2 = jnp.stack([ids[::R], ids[R - 1::R]], axis=-1).reshape(nblk, 1, 2)
    b2 = b.reshape(1, D)
    return pl.pallas_call(
        _body,
        grid=(nblk,),
        in_specs=[
            pl.BlockSpec((R, D), lambda i: (i, 0)),
            pl.BlockSpec((1, 1, R), lambda i: (i, 0, 0)),
            pl.BlockSpec((1, 1, 2), lambda i: (i, 0, 0),
                         memory_space=pltpu.SMEM),
            pl.BlockSpec((D, D), lambda i: (0, 0)),
            pl.BlockSpec((1, D), lambda i: (0, 0)),
        ],
        out_specs=pl.BlockSpec((NSEG, 2 * D), lambda i: (0, 0)),
        out_shape=jax.ShapeDtypeStruct((NSEG, 2 * D), jnp.float32),
        scratch_shapes=[pltpu.VMEM((ACC_ROWS, 3 * D), jnp.float32)],
        compiler_params=pltpu.CompilerParams(
            dimension_semantics=("arbitrary",)),
    )(x, ids3, ids2, W, b2)


# speedup vs baseline: 1.5130x; 1.0391x over previous
import jax
import jax.numpy as jnp
from jax.experimental import pallas as pl
from jax.experimental.pallas import tpu as pltpu

NSEG = 10000
SWIN = 96
ACC_ROWS = 10112


def _pick_block_rows(n):
    for r in (2560, 1280, 640, 320, 160, 80, 40, 8):
        if n % r == 0:
            return r
    return n


def _body(x_ref, ids_v_ref, ids_s_ref, w_ref, b_ref, out_ref, acc_ref):
    i = pl.program_id(0)
    nblk = pl.num_programs(0)
    R = x_ref.shape[0]
    D = x_ref.shape[1]

    @pl.when(i == 0)
    def _init():
        acc_ref[...] = jnp.zeros_like(acc_ref)

    feat = jax.lax.dot_general(
        x_ref[...].astype(jnp.bfloat16), w_ref[...].astype(jnp.bfloat16),
        (((1,), (1,)), ((), ())),
        preferred_element_type=jnp.float32)
    feat = feat + b_ref[...]
    featcat = jnp.concatenate(
        [feat, feat * feat, jnp.ones((R, D), jnp.float32)],
        axis=1).astype(jnp.bfloat16)

    ids_v = ids_v_ref[0]
    s_lo = ids_s_ref[0, 0, 0]
    s_hi = ids_s_ref[0, 0, 1]
    a0 = (s_lo // 8) * 8

    def win_body(base):
        base = pl.multiple_of(base, 8)
        rel = ids_v - base
        sidx = jax.lax.broadcasted_iota(jnp.int32, (SWIN, 1), 0)
        P = jnp.where(rel == sidx, 1.0, 0.0)
        part = jax.lax.dot_general(
            P.astype(jnp.bfloat16), featcat, (((1,), (0,)), ((), ())),
            preferred_element_type=jnp.float32)
        acc_ref[pl.ds(base, SWIN), :] += part
        return base + SWIN

    win_body(a0)

    @pl.when(s_hi >= a0 + SWIN)
    def _second():
        win_body(a0 + SWIN)

    @pl.when(s_hi >= a0 + 2 * SWIN)
    def _rest():
        jax.lax.while_loop(lambda b: b <= s_hi, win_body, a0 + 2 * SWIN)

    @pl.when(i == nblk - 1)
    def _fin():
        sums = acc_ref[:NSEG, 0:D]
        sumsq = acc_ref[:NSEG, D:2 * D]
        cnt = acc_ref[:NSEG, 2 * D:2 * D + 1]
        cnt_c = jnp.maximum(cnt, 1.0)
        mean = sums / cnt_c
        sq = jnp.maximum(sumsq - sums * mean, 0.0)
        cnt_u = jnp.maximum(cnt - 1.0, 1.0)
        std = jnp.sqrt(sq / (cnt_u + 1e-6))
        out_ref[:, 0:D] = mean
        out_ref[:, D:2 * D] = std


def kernel(x, batch, W, b):
    N, D = x.shape
    R = _pick_block_rows(N)
    nblk = N // R
    ids = batch.astype(jnp.int32)
    ids3 = ids.reshape(nblk, 1, R)
    ids2 = jnp.stack([ids[::R], ids[R - 1::R]], axis=-1).reshape(nblk, 1, 2)
    b2 = b.reshape(1, D)
    return pl.pallas_call(
        _body,
        grid=(nblk,),
        in_specs=[
            pl.BlockSpec((R, D), lambda i: (i, 0)),
            pl.BlockSpec((1, 1, R), lambda i: (i, 0, 0)),
            pl.BlockSpec((1, 1, 2), lambda i: (i, 0, 0),
                         memory_space=pltpu.SMEM),
            pl.BlockSpec((D, D), lambda i: (0, 0)),
            pl.BlockSpec((1, D), lambda i: (0, 0)),
        ],
        out_specs=pl.BlockSpec((NSEG, 2 * D), lambda i: (0, 0)),
        out_shape=jax.ShapeDtypeStruct((NSEG, 2 * D), jnp.float32),
        scratch_shapes=[pltpu.VMEM((ACC_ROWS, 3 * D), jnp.float32)],
        compiler_params=pltpu.CompilerParams(
            dimension_semantics=("arbitrary",)),
    )(x, ids3, ids2, W, b2)


# speedup vs baseline: 1.7327x; 1.1453x over previous
import jax
import jax.numpy as jnp
from jax.experimental import pallas as pl
from jax.experimental.pallas import tpu as pltpu

NSEG = 10000
SWIN = 160
ACC_ROWS = 10176


def _pick_block_rows(n):
    for r in (4000, 2560, 1280, 640, 320, 160, 80, 40, 8):
        if n % r == 0:
            return r
    return n


def _body(x_ref, ids_v_ref, ids_s_ref, w_ref, b_ref, out_ref, acc_ref):
    i = pl.program_id(0)
    nblk = pl.num_programs(0)
    R = x_ref.shape[0]
    D = x_ref.shape[1]

    @pl.when(i == 0)
    def _init():
        acc_ref[...] = jnp.zeros_like(acc_ref)

    feat = jax.lax.dot_general(
        x_ref[...].astype(jnp.bfloat16), w_ref[...].astype(jnp.bfloat16),
        (((1,), (1,)), ((), ())),
        preferred_element_type=jnp.float32)
    feat = feat + b_ref[...]
    featcat = jnp.concatenate(
        [feat, feat * feat, jnp.ones((R, D), jnp.float32)],
        axis=1).astype(jnp.bfloat16)

    ids_v = ids_v_ref[0]
    s_lo = ids_s_ref[0, 0, 0]
    s_hi = ids_s_ref[0, 0, 1]
    a0 = (s_lo // 8) * 8

    def win_body(base):
        base = pl.multiple_of(base, 8)
        rel = ids_v - base
        sidx = jax.lax.broadcasted_iota(jnp.int32, (SWIN, 1), 0)
        P = jnp.where(rel == sidx, 1.0, 0.0)
        part = jax.lax.dot_general(
            P.astype(jnp.bfloat16), featcat, (((1,), (0,)), ((), ())),
            preferred_element_type=jnp.float32)
        acc_ref[pl.ds(base, SWIN), :] += part
        return base + SWIN

    win_body(a0)

    @pl.when(s_hi >= a0 + SWIN)
    def _second():
        win_body(a0 + SWIN)

    @pl.when(s_hi >= a0 + 2 * SWIN)
    def _rest():
        jax.lax.while_loop(lambda b: b <= s_hi, win_body, a0 + 2 * SWIN)

    @pl.when(i == nblk - 1)
    def _fin():
        sums = acc_ref[:NSEG, 0:D]
        sumsq = acc_ref[:NSEG, D:2 * D]
        cnt = acc_ref[:NSEG, 2 * D:2 * D + 1]
        cnt_c = jnp.maximum(cnt, 1.0)
        mean = sums / cnt_c
        sq = jnp.maximum(sumsq - sums * mean, 0.0)
        cnt_u = jnp.maximum(cnt - 1.0, 1.0)
        std = jnp.sqrt(sq / (cnt_u + 1e-6))
        out_ref[:, 0:D] = mean
        out_ref[:, D:2 * D] = std


def kernel(x, batch, W, b):
    N, D = x.shape
    R = _pick_block_rows(N)
    nblk = N // R
    ids = batch.astype(jnp.int32)
    ids3 = ids.reshape(nblk, 1, R)
    ids2 = jnp.stack([ids[::R], ids[R - 1::R]], axis=-1).reshape(nblk, 1, 2)
    b2 = b.reshape(1, D)
    return pl.pallas_call(
        _body,
        grid=(nblk,),
        in_specs=[
            pl.BlockSpec((R, D), lambda i: (i, 0)),
            pl.BlockSpec((1, 1, R), lambda i: (i, 0, 0)),
            pl.BlockSpec((1, 1, 2), lambda i: (i, 0, 0),
                         memory_space=pltpu.SMEM),
            pl.BlockSpec((D, D), lambda i: (0, 0)),
            pl.BlockSpec((1, D), lambda i: (0, 0)),
        ],
        out_specs=pl.BlockSpec((NSEG, 2 * D), lambda i: (0, 0)),
        out_shape=jax.ShapeDtypeStruct((NSEG, 2 * D), jnp.float32),
        scratch_shapes=[pltpu.VMEM((ACC_ROWS, 3 * D), jnp.float32)],
        compiler_params=pltpu.CompilerParams(
            dimension_semantics=("arbitrary",)),
    )(x, ids3, ids2, W, b2)
